# Initial kernel scaffold; baseline (speedup 1.0000x reference)
import jax
import jax.numpy as jnp
from jax.experimental import pallas as pl


def kernel(class_ids, coords, emb_table, W1, b1, W2, b2):
    raise NotImplementedError("write your pallas kernel here")


# speedup vs baseline: 1.0812x; 1.0812x over previous
import functools

import jax
import jax.numpy as jnp
from jax import lax
from jax.experimental import pallas as pl
from jax.experimental.pallas import tpu as pltpu
from jax.experimental.pallas import tpu_sc as plsc

NUM_CLASSES = 1000
HALF = 64
OUT_DIM = 2 * HALF
B, N = 4096, 200
TOT = B * N

NC, NS = 2, 16
NW = NC * NS
PER_W = TOT // NW
CH = 128
N_CHUNKS = PER_W // CH


def _sc_gather(ids_flat, table):
    mesh = plsc.VectorSubcoreMesh(core_axis_name="c", subcore_axis_name="s")

    @functools.partial(
        pl.kernel,
        out_type=jax.ShapeDtypeStruct((TOT, HALF), jnp.float32),
        mesh=mesh,
        compiler_params=pltpu.CompilerParams(use_tc_tiling_on_sc=False),
        scratch_types=[
            pltpu.VMEM((PER_W,), jnp.int32),
            pltpu.VMEM((CH, HALF), jnp.float32),
            pltpu.SemaphoreType.DMA,
        ],
    )
    def sc_body(ids_hbm, table_hbm, out_hbm, idx_v, rows_v, sem):
        wid = lax.axis_index("s") * NC + lax.axis_index("c")
        base = wid * PER_W
        pltpu.sync_copy(ids_hbm.at[pl.ds(base, PER_W)], idx_v)

        @pl.loop(0, N_CHUNKS)
        def _chunk(c):
            pltpu.async_copy(
                table_hbm.at[idx_v.at[pl.ds(c * CH, CH)]], rows_v, sem
            ).wait()
            pltpu.sync_copy(rows_v, out_hbm.at[pl.ds(base + c * CH, CH)])

    return sc_body(ids_flat, table)


BLK = 8192


def _tc_body(class_ref, coords_ref, w1_ref, b1_ref, w2_ref, b2_ref, out_ref):
    c = coords_ref[...]
    w1 = w1_ref[...]
    h = (
        c[:, 0:1] * w1[0:1, :]
        + c[:, 1:2] * w1[1:2, :]
        + c[:, 2:3] * w1[2:3, :]
        + b1_ref[...]
    )
    h = jnp.maximum(h, 0.0)
    coord_emb = (
        jax.lax.dot_general(
            h, w2_ref[...], (((1,), (0,)), ((), ())),
            preferred_element_type=jnp.float32,
        )
        + b2_ref[...]
    )
    out_ref[:, :HALF] = class_ref[...]
    out_ref[:, HALF:] = coord_emb


def _tc_mlp(class_half, coords_flat, W1, b1, W2, b2):
    grid = (TOT // BLK,)
    return pl.pallas_call(
        _tc_body,
        grid=grid,
        in_specs=[
            pl.BlockSpec((BLK, HALF), lambda i: (i, 0)),
            pl.BlockSpec((BLK, 3), lambda i: (i, 0)),
            pl.BlockSpec((3, HALF), lambda i: (0, 0)),
            pl.BlockSpec((1, HALF), lambda i: (0, 0)),
            pl.BlockSpec((HALF, HALF), lambda i: (0, 0)),
            pl.BlockSpec((1, HALF), lambda i: (0, 0)),
        ],
        out_specs=pl.BlockSpec((BLK, OUT_DIM), lambda i: (i, 0)),
        out_shape=jax.ShapeDtypeStruct((TOT, OUT_DIM), jnp.float32),
    )(class_half, coords_flat, W1, b1, W2, b2)


def kernel(class_ids, coords, emb_table, W1, b1, W2, b2):
    ids_flat = class_ids.reshape(TOT).astype(jnp.int32)
    coords_flat = coords.reshape(TOT, 3)
    class_half = _sc_gather(ids_flat, emb_table)
    out = _tc_mlp(
        class_half, coords_flat, W1, b1.reshape(1, HALF), W2, b2.reshape(1, HALF)
    )
    return out.reshape(B, N, OUT_DIM)


# speedup vs baseline: 1.2004x; 1.1102x over previous
import functools

import jax
import jax.numpy as jnp
from jax import lax
from jax.experimental import pallas as pl
from jax.experimental.pallas import tpu as pltpu
from jax.experimental.pallas import tpu_sc as plsc

NUM_CLASSES = 1000
HALF = 64
OUT_DIM = 2 * HALF
B, N = 4096, 200
TOT = B * N

NC, NS = 2, 16
NW = NC * NS
PER_W = TOT // NW
CH = 128
N_CHUNKS = PER_W // CH


K = 4
GRP = K * CH
N_GRP = PER_W // GRP


def _sc_gather(ids_flat, table):
    mesh = plsc.VectorSubcoreMesh(core_axis_name="c", subcore_axis_name="s")

    @functools.partial(
        pl.kernel,
        out_type=jax.ShapeDtypeStruct((TOT, HALF), jnp.float32),
        mesh=mesh,
        compiler_params=pltpu.CompilerParams(use_tc_tiling_on_sc=False),
        scratch_types=[
            pltpu.VMEM((PER_W,), jnp.int32),
            pltpu.VMEM((GRP, HALF), jnp.float32),
            pltpu.VMEM((GRP, HALF), jnp.float32),
            pltpu.VMEM_SHARED((NUM_CLASSES, HALF), jnp.float32),
            pltpu.SemaphoreType.DMA,
            pltpu.SemaphoreType.DMA,
            pltpu.SemaphoreType.DMA,
            pltpu.SemaphoreType.DMA,
        ],
    )
    def sc_body(ids_hbm, table_hbm, out_hbm, idx_v, rows0, rows1, tab_s,
                g0, g1, w0, w1):
        cid = lax.axis_index("c")
        sid = lax.axis_index("s")
        wid = sid * NC + cid
        base = wid * PER_W

        @pl.when(sid == 0)
        def _stage_table():
            pltpu.sync_copy(table_hbm, tab_s)

        plsc.subcore_barrier()
        pltpu.sync_copy(ids_hbm.at[pl.ds(base, PER_W)], idx_v)

        def issue_gathers(g, rows, gsem):
            for j in range(K):
                pltpu.async_copy(
                    tab_s.at[idx_v.at[pl.ds(g * GRP + j * CH, CH)]],
                    rows.at[pl.ds(j * CH, CH)],
                    gsem,
                )

        def drain_gathers(rows, gsem):
            pltpu.make_async_copy(tab_s.at[pl.ds(0, GRP)], rows, gsem).wait()

        def issue_write(g, rows, wsem):
            pltpu.async_copy(rows, out_hbm.at[pl.ds(base + g * GRP, GRP)], wsem)

        def drain_write(rows, wsem):
            pltpu.make_async_copy(rows, out_hbm.at[pl.ds(0, GRP)], wsem).wait()

        issue_gathers(0, rows0, g0)
        issue_gathers(1, rows1, g1)

        @pl.loop(0, N_GRP, step=2)
        def _grp(g):
            drain_gathers(rows0, g0)
            issue_write(g, rows0, w0)
            drain_gathers(rows1, g1)
            issue_write(g + 1, rows1, w1)

            @pl.when(g + 2 < N_GRP)
            def _refill0():
                drain_write(rows0, w0)
                issue_gathers(g + 2, rows0, g0)

            @pl.when(g + 3 < N_GRP)
            def _refill1():
                drain_write(rows1, w1)
                issue_gathers(g + 3, rows1, g1)

        drain_write(rows0, w0)
        drain_write(rows1, w1)

    return sc_body(ids_flat, table)


BLK = 8192


def _tc_body(class_ref, coords_ref, w1_ref, b1_ref, w2_ref, b2_ref, out_ref):
    c = coords_ref[...]
    h = (
        jax.lax.dot_general(
            c, w1_ref[...], (((1,), (0,)), ((), ())),
            preferred_element_type=jnp.float32,
        )
        + b1_ref[...]
    )
    h = jnp.maximum(h, 0.0)
    coord_emb = (
        jax.lax.dot_general(
            h, w2_ref[...], (((1,), (0,)), ((), ())),
            preferred_element_type=jnp.float32,
        )
        + b2_ref[...]
    )
    out_ref[...] = jnp.concatenate([class_ref[...], coord_emb], axis=1)


def _tc_mlp(class_half, coords_flat, W1, b1, W2, b2):
    grid = (TOT // BLK,)
    return pl.pallas_call(
        _tc_body,
        grid=grid,
        in_specs=[
            pl.BlockSpec((BLK, HALF), lambda i: (i, 0)),
            pl.BlockSpec((BLK, 3), lambda i: (i, 0)),
            pl.BlockSpec((3, HALF), lambda i: (0, 0)),
            pl.BlockSpec((1, HALF), lambda i: (0, 0)),
            pl.BlockSpec((HALF, HALF), lambda i: (0, 0)),
            pl.BlockSpec((1, HALF), lambda i: (0, 0)),
        ],
        out_specs=pl.BlockSpec((BLK, OUT_DIM), lambda i: (i, 0)),
        out_shape=jax.ShapeDtypeStruct((TOT, OUT_DIM), jnp.float32),
    )(class_half, coords_flat, W1, b1, W2, b2)


def kernel(class_ids, coords, emb_table, W1, b1, W2, b2):
    ids_flat = class_ids.reshape(TOT).astype(jnp.int32)
    coords_flat = coords.reshape(TOT, 3)
    class_half = _sc_gather(ids_flat, emb_table)
    out = _tc_mlp(
        class_half, coords_flat, W1, b1.reshape(1, HALF), W2, b2.reshape(1, HALF)
    )
    return out.reshape(B, N, OUT_DIM)
